# baseline (speedup 1.0000x reference)
import functools

import jax
import jax.numpy as jnp
from jax import lax
from jax.experimental import pallas as pl
from jax.experimental.pallas import tpu as pltpu
from jax.experimental.pallas import tpu_sc as plsc

_N = 10000
_E = 320000
_D_IN = 128
_D_HID = 128
_D_OUT = 64
_TAU = 0.67

_NW = 32
_C = 50
_K = _E // (_NW * _C)
_G = 20
_NSEG = _K // _G
_NPAD = 10240
_RPT = _NPAD // 16
_DPT = _NPAD // 16

_mesh = plsc.VectorSubcoreMesh(core_axis_name="c", subcore_axis_name="s")


_KD = 125
_CD = 80


@functools.partial(
    pl.kernel,
    out_type=jax.ShapeDtypeStruct((2 * _NPAD,), jnp.float32),
    mesh=_mesh,
    scratch_types=[
        pltpu.VMEM((_KD, _CD), jnp.int32),
        pltpu.VMEM((_CD,), jnp.float32),
        pltpu.VMEM((_DPT,), jnp.float32),
        pltpu.VMEM_SHARED((_NPAD,), jnp.float32),
        pltpu.SemaphoreType.DMA,
    ],
)
def _deg_kernel(dst_hbm, out_hbm, didx, ones_v, zbuf, acc, dsem):
    cid = lax.axis_index("c")
    sid = lax.axis_index("s")
    wid = cid * 16 + sid

    def fill_ones(i, _):
        ones_v[pl.ds(i * 16, 16)] = jnp.ones((16,), jnp.float32)
        return 0

    lax.fori_loop(0, _CD // 16, fill_ones, 0)

    def fill_zero(i, _):
        zbuf[pl.ds(i * 16, 16)] = jnp.zeros((16,), jnp.float32)
        return 0

    lax.fori_loop(0, _DPT // 16, fill_zero, 0)

    pltpu.sync_copy(zbuf, acc.at[pl.ds(sid * _DPT, _DPT)])
    pltpu.sync_copy(dst_hbm.at[wid], didx)
    plsc.subcore_barrier()

    _B = 25

    def batch(b, _):
        def fire(j, _):
            pltpu.async_copy(ones_v, acc.at[didx.at[b * _B + j]], dsem,
                             add=True)
            return 0

        lax.fori_loop(0, _B, fire, 0)

        def drain(j, _):
            pltpu.make_async_copy(ones_v, acc.at[didx.at[b * _B + j]],
                                  dsem).wait()
            return 0

        lax.fori_loop(0, _B, drain, 0)
        return 0

    lax.fori_loop(0, _KD // _B, batch, 0)
    plsc.subcore_barrier()
    pltpu.sync_copy(acc.at[pl.ds(sid * _DPT, _DPT)],
                    out_hbm.at[pl.ds(cid * _NPAD + sid * _DPT, _DPT)])


def _make_conv(d, c, k, g):
    nseg = k // g

    @functools.partial(
        pl.kernel,
        out_type=jax.ShapeDtypeStruct((2, _NPAD, d), jnp.float32),
        mesh=_mesh,
        compiler_params=pltpu.CompilerParams(use_tc_tiling_on_sc=False),
        scratch_types=[
            pltpu.VMEM((2, g, 2, c), jnp.int32),
            pltpu.VMEM((4, c, d), jnp.float32),
            pltpu.VMEM_SHARED((_NPAD, d), jnp.float32),
            pltpu.SemaphoreType.DMA,
            pltpu.SemaphoreType.DMA,
            pltpu.SemaphoreType.DMA,
            pltpu.SemaphoreType.DMA,
            pltpu.SemaphoreType.DMA,
            pltpu.SemaphoreType.DMA,
            pltpu.SemaphoreType.DMA,
            pltpu.SemaphoreType.DMA,
            pltpu.SemaphoreType.DMA,
            pltpu.SemaphoreType.DMA,
        ],
    )
    def _conv(u_hbm, edges_hbm, out_hbm, seg, rows, acc, gs0, gs1, gs2, gs3,
              ss0, ss1, ss2, ss3, is0, is1):
        cid = lax.axis_index("c")
        sid = lax.axis_index("s")
        wid = cid * 16 + sid
        nvec = d // 16
        gsems = (gs0, gs1, gs2, gs3)
        ssems = (ss0, ss1, ss2, ss3)
        isems = (is0, is1)

        def fill_zero(i, _):
            r = i // nvec
            col = i % nvec
            rows[0, r, pl.ds(col * 16, 16)] = jnp.zeros((16,), jnp.float32)
            return 0

        lax.fori_loop(0, c * nvec, fill_zero, 0)

        base = sid * _RPT
        nz = _RPT // c
        for z in range(nz):
            pltpu.sync_copy(rows.at[0, pl.ds(0, c)],
                            acc.at[pl.ds(base + z * c, c)])
        rem = _RPT - nz * c
        if rem:
            pltpu.sync_copy(rows.at[0, pl.ds(0, rem)],
                            acc.at[pl.ds(base + nz * c, rem)])

        pltpu.sync_copy(edges_hbm.at[wid, pl.ds(0, g)], seg.at[0])
        plsc.subcore_barrier()
        pltpu.async_copy(u_hbm.at[seg.at[0, 0, 0]], rows.at[0], gs0)

        for s in range(nseg):
            sslot = s % 2
            nslot = (s + 1) % 2
            if s + 1 < nseg:
                pltpu.async_copy(edges_hbm.at[wid, pl.ds((s + 1) * g, g)],
                                 seg.at[nslot], isems[nslot])

            def proc(gg, r):
                rw = (r + 1) % 4

                @pl.when(gg >= 3)
                def _():
                    pltpu.make_async_copy(
                        rows.at[rw], acc.at[seg.at[sslot, gg - 3, 1]],
                        ssems[rw]).wait()

                if s > 0:
                    @pl.when(gg < 3)
                    def _():
                        pltpu.make_async_copy(
                            rows.at[rw], acc.at[seg.at[nslot, gg + g - 3, 1]],
                            ssems[rw]).wait()

                @pl.when(gg + 1 < g)
                def _():
                    pltpu.async_copy(u_hbm.at[seg.at[sslot, gg + 1, 0]],
                                     rows.at[rw], gsems[rw])

                if s + 1 < nseg:
                    @pl.when(gg + 1 == g)
                    def _():
                        pltpu.make_async_copy(
                            edges_hbm.at[wid, pl.ds((s + 1) * g, g)],
                            seg.at[nslot], isems[nslot]).wait()
                        pltpu.async_copy(u_hbm.at[seg.at[nslot, 0, 0]],
                                         rows.at[rw], gsems[rw])

                pltpu.make_async_copy(u_hbm.at[seg.at[sslot, gg, 0]],
                                      rows.at[r], gsems[r]).wait()
                pltpu.async_copy(rows.at[r], acc.at[seg.at[sslot, gg, 1]],
                                 ssems[r], add=True)

            def inner(t, _):
                proc(4 * t, 0)
                proc(4 * t + 1, 1)
                proc(4 * t + 2, 2)
                proc(4 * t + 3, 3)
                return 0

            lax.fori_loop(0, g // 4, inner, 0)

        ls = (nseg - 1) % 2
        pltpu.make_async_copy(rows.at[1], acc.at[seg.at[ls, g - 3, 1]],
                              ssems[1]).wait()
        pltpu.make_async_copy(rows.at[2], acc.at[seg.at[ls, g - 2, 1]],
                              ssems[2]).wait()
        pltpu.make_async_copy(rows.at[3], acc.at[seg.at[ls, g - 1, 1]],
                              ssems[3]).wait()
        plsc.subcore_barrier()
        pltpu.sync_copy(acc.at[pl.ds(base, _RPT)],
                        out_hbm.at[cid, pl.ds(base, _RPT)])

    return _conv


_C64 = 125
_K64 = _E // (_NW * _C64)
_G64 = 16

_conv128 = _make_conv(_D_HID, _C, _K, _G)
_conv64 = _make_conv(_D_OUT, _C64, _K64, _G64)


_R = 1000


def _pre_body(x_ref, wg1_ref, d0_ref, d1_ref, u1_ref, dinv_ref):
    x = x_ref[...]
    deg = d0_ref[0, 0, :] + d1_ref[0, 0, :] + 1.0
    dinv = lax.rsqrt(deg)
    dinv_ref[0, 0, :] = dinv
    u1_ref[...] = jnp.dot(x, wg1_ref[...],
                          preferred_element_type=jnp.float32) * dinv[:, None]


_pre_call = pl.pallas_call(
    _pre_body,
    grid=(_N // _R,),
    in_specs=[
        pl.BlockSpec((_R, _D_IN), lambda i: (i, 0)),
        pl.BlockSpec((_D_IN, _D_HID), lambda i: (0, 0)),
        pl.BlockSpec((1, 1, _R), lambda i: (i, 0, 0)),
        pl.BlockSpec((1, 1, _R), lambda i: (i, 0, 0)),
    ],
    out_specs=[
        pl.BlockSpec((_R, _D_HID), lambda i: (i, 0)),
        pl.BlockSpec((1, 1, _R), lambda i: (i, 0, 0)),
    ],
    out_shape=[
        jax.ShapeDtypeStruct((_N, _D_HID), jnp.float32),
        jax.ShapeDtypeStruct((_N // _R, 1, _R), jnp.float32),
    ],
)


def _mid_body(s1a_ref, s1b_ref, u1_ref, dinv_ref, bg1_ref, wg2_ref, u2_ref):
    dinv = dinv_ref[0, 0, :]
    z = (s1a_ref[...] + s1b_ref[...] + u1_ref[...]) * dinv[:, None] + bg1_ref[...][None, :]
    z = jnp.maximum(z, 0.0)
    u2_ref[...] = jnp.dot(z, wg2_ref[...],
                          preferred_element_type=jnp.float32) * dinv[:, None]


_mid_call = pl.pallas_call(
    _mid_body,
    grid=(_N // _R,),
    in_specs=[
        pl.BlockSpec((_R, _D_HID), lambda i: (i, 0)),
        pl.BlockSpec((_R, _D_HID), lambda i: (i, 0)),
        pl.BlockSpec((_R, _D_HID), lambda i: (i, 0)),
        pl.BlockSpec((1, 1, _R), lambda i: (i, 0, 0)),
        pl.BlockSpec((_D_HID,), lambda i: (0,)),
        pl.BlockSpec((_D_HID, _D_OUT), lambda i: (0, 0)),
    ],
    out_specs=pl.BlockSpec((_R, _D_OUT), lambda i: (i, 0)),
    out_shape=jax.ShapeDtypeStruct((_N, _D_OUT), jnp.float32),
)


def _post_body(s2a_ref, s2b_ref, u2_ref, dinv_ref, bg2_ref, x_ref, wm1_ref,
               bm1_ref, wm2_ref, bm2_ref, spi_ref, logt_ref, out_ref):
    dinv = dinv_ref[0, 0, :]
    z_gnn = ((s2a_ref[...] + s2b_ref[...] + u2_ref[...]) * dinv[:, None]
             + bg2_ref[...][None, :])
    m = jnp.maximum(jnp.dot(x_ref[...], wm1_ref[...],
                            preferred_element_type=jnp.float32)
                    + bm1_ref[...][None, :], 0.0)
    z_mlp = jnp.dot(m, wm2_ref[...],
                    preferred_element_type=jnp.float32) + bm2_ref[...][None, :]
    beta = jax.nn.sigmoid((spi_ref[0, 0] - _TAU) * jnp.exp(-logt_ref[0, 0]))
    out_ref[...] = beta * z_gnn + (1.0 - beta) * z_mlp


_post_call = pl.pallas_call(
    _post_body,
    grid=(_N // _R,),
    in_specs=[
        pl.BlockSpec((_R, _D_OUT), lambda i: (i, 0)),
        pl.BlockSpec((_R, _D_OUT), lambda i: (i, 0)),
        pl.BlockSpec((_R, _D_OUT), lambda i: (i, 0)),
        pl.BlockSpec((1, 1, _R), lambda i: (i, 0, 0)),
        pl.BlockSpec((_D_OUT,), lambda i: (0,)),
        pl.BlockSpec((_R, _D_IN), lambda i: (i, 0)),
        pl.BlockSpec((_D_IN, _D_HID), lambda i: (0, 0)),
        pl.BlockSpec((_D_HID,), lambda i: (0,)),
        pl.BlockSpec((_D_HID, _D_OUT), lambda i: (0, 0)),
        pl.BlockSpec((_D_OUT,), lambda i: (0,)),
        pl.BlockSpec((1, 1), lambda i: (0, 0)),
        pl.BlockSpec((1, 1), lambda i: (0, 0)),
    ],
    out_specs=pl.BlockSpec((_R, _D_OUT), lambda i: (i, 0)),
    out_shape=jax.ShapeDtypeStruct((_N, _D_OUT), jnp.float32),
)


def kernel(x, edge_index, spi, W_g1, b_g1, W_g2, b_g2, W_m1, b_m1, W_m2, b_m2, log_T):
    srcr = edge_index[0].reshape(_NW, _K, _C)
    dstr = edge_index[1].reshape(_NW, _K, _C)
    edges = jnp.stack([srcr, dstr], axis=2)
    src64 = edge_index[0].reshape(_NW, _K64, _C64)
    dst64 = edge_index[1].reshape(_NW, _K64, _C64)
    edges64 = jnp.stack([src64, dst64], axis=2)
    dstd = edge_index[1].reshape(_NW, _KD, _CD)
    degp = _deg_kernel(dstd)
    d0 = degp[:_N].reshape(_N // _R, 1, _R)
    d1 = degp[_NPAD:_NPAD + _N].reshape(_N // _R, 1, _R)
    u1, dinv = _pre_call(x, W_g1, d0, d1)
    s1 = _conv128(u1, edges)
    u2 = _mid_call(s1[0, :_N], s1[1, :_N], u1, dinv, b_g1, W_g2)
    s2 = _conv64(u2, edges64)
    out = _post_call(s2[0, :_N], s2[1, :_N], u2, dinv, b_g2, x, W_m1, b_m1,
                     W_m2, b_m2, spi.reshape(1, 1), log_T.reshape(1, 1))
    return out


# speedup vs baseline: 1.0043x; 1.0043x over previous
import functools

import jax
import jax.numpy as jnp
from jax import lax
from jax.experimental import pallas as pl
from jax.experimental.pallas import tpu as pltpu
from jax.experimental.pallas import tpu_sc as plsc

_N = 10000
_E = 320000
_D_IN = 128
_D_HID = 128
_D_OUT = 64
_TAU = 0.67

_NW = 32
_C = 50
_K = _E // (_NW * _C)
_G = 20
_NSEG = _K // _G
_NPAD = 10240
_RPT = _NPAD // 16
_DPT = _NPAD // 16

_mesh = plsc.VectorSubcoreMesh(core_axis_name="c", subcore_axis_name="s")


_KD = 125
_CD = 80


@functools.partial(
    pl.kernel,
    out_type=jax.ShapeDtypeStruct((2 * _NPAD,), jnp.float32),
    mesh=_mesh,
    scratch_types=[
        pltpu.VMEM((_KD, _CD), jnp.int32),
        pltpu.VMEM((_CD,), jnp.float32),
        pltpu.VMEM((_DPT,), jnp.float32),
        pltpu.VMEM_SHARED((_NPAD,), jnp.float32),
    ],
)
def _deg_kernel(dst_hbm, out_hbm, didx, ones_v, zbuf, acc):
    cid = lax.axis_index("c")
    sid = lax.axis_index("s")
    wid = cid * 16 + sid

    def fill_ones(i, _):
        ones_v[pl.ds(i * 16, 16)] = jnp.ones((16,), jnp.float32)
        return 0

    lax.fori_loop(0, _CD // 16, fill_ones, 0)

    def fill_zero(i, _):
        zbuf[pl.ds(i * 16, 16)] = jnp.zeros((16,), jnp.float32)
        return 0

    lax.fori_loop(0, _DPT // 16, fill_zero, 0)

    pltpu.sync_copy(zbuf, acc.at[pl.ds(sid * _DPT, _DPT)])
    pltpu.sync_copy(dst_hbm.at[wid], didx)
    plsc.subcore_barrier()

    def body(j, _):
        pltpu.sync_copy(ones_v, acc.at[didx.at[j]], add=True)
        return 0

    lax.fori_loop(0, _KD, body, 0)
    plsc.subcore_barrier()
    pltpu.sync_copy(acc.at[pl.ds(sid * _DPT, _DPT)],
                    out_hbm.at[pl.ds(cid * _NPAD + sid * _DPT, _DPT)])


def _make_conv(d, c, k, g):
    nseg = k // g

    @functools.partial(
        pl.kernel,
        out_type=jax.ShapeDtypeStruct((2, _NPAD, d), jnp.float32),
        mesh=_mesh,
        compiler_params=pltpu.CompilerParams(use_tc_tiling_on_sc=False),
        scratch_types=[
            pltpu.VMEM((2, g, 2, c), jnp.int32),
            pltpu.VMEM((4, c, d), jnp.float32),
            pltpu.VMEM_SHARED((_NPAD, d), jnp.float32),
            pltpu.SemaphoreType.DMA,
            pltpu.SemaphoreType.DMA,
            pltpu.SemaphoreType.DMA,
            pltpu.SemaphoreType.DMA,
            pltpu.SemaphoreType.DMA,
            pltpu.SemaphoreType.DMA,
            pltpu.SemaphoreType.DMA,
            pltpu.SemaphoreType.DMA,
            pltpu.SemaphoreType.DMA,
            pltpu.SemaphoreType.DMA,
        ],
    )
    def _conv(u_hbm, edges_hbm, out_hbm, seg, rows, acc, gs0, gs1, gs2, gs3,
              ss0, ss1, ss2, ss3, is0, is1):
        cid = lax.axis_index("c")
        sid = lax.axis_index("s")
        wid = cid * 16 + sid
        nvec = d // 16
        gsems = (gs0, gs1, gs2, gs3)
        ssems = (ss0, ss1, ss2, ss3)
        isems = (is0, is1)

        def fill_zero(i, _):
            r = i // nvec
            col = i % nvec
            rows[0, r, pl.ds(col * 16, 16)] = jnp.zeros((16,), jnp.float32)
            return 0

        lax.fori_loop(0, c * nvec, fill_zero, 0)

        base = sid * _RPT
        nz = _RPT // c
        for z in range(nz):
            pltpu.sync_copy(rows.at[0, pl.ds(0, c)],
                            acc.at[pl.ds(base + z * c, c)])
        rem = _RPT - nz * c
        if rem:
            pltpu.sync_copy(rows.at[0, pl.ds(0, rem)],
                            acc.at[pl.ds(base + nz * c, rem)])

        pltpu.sync_copy(edges_hbm.at[wid, pl.ds(0, g)], seg.at[0])
        plsc.subcore_barrier()
        pltpu.async_copy(u_hbm.at[seg.at[0, 0, 0]], rows.at[0], gs0)

        for s in range(nseg):
            sslot = s % 2
            nslot = (s + 1) % 2
            if s + 1 < nseg:
                pltpu.async_copy(edges_hbm.at[wid, pl.ds((s + 1) * g, g)],
                                 seg.at[nslot], isems[nslot])

            def proc(gg, r):
                rw = (r + 1) % 4

                @pl.when(gg >= 3)
                def _():
                    pltpu.make_async_copy(
                        rows.at[rw], acc.at[seg.at[sslot, gg - 3, 1]],
                        ssems[rw]).wait()

                if s > 0:
                    @pl.when(gg < 3)
                    def _():
                        pltpu.make_async_copy(
                            rows.at[rw], acc.at[seg.at[nslot, gg + g - 3, 1]],
                            ssems[rw]).wait()

                @pl.when(gg + 1 < g)
                def _():
                    pltpu.async_copy(u_hbm.at[seg.at[sslot, gg + 1, 0]],
                                     rows.at[rw], gsems[rw])

                if s + 1 < nseg:
                    @pl.when(gg + 1 == g)
                    def _():
                        pltpu.make_async_copy(
                            edges_hbm.at[wid, pl.ds((s + 1) * g, g)],
                            seg.at[nslot], isems[nslot]).wait()
                        pltpu.async_copy(u_hbm.at[seg.at[nslot, 0, 0]],
                                         rows.at[rw], gsems[rw])

                pltpu.make_async_copy(u_hbm.at[seg.at[sslot, gg, 0]],
                                      rows.at[r], gsems[r]).wait()
                pltpu.async_copy(rows.at[r], acc.at[seg.at[sslot, gg, 1]],
                                 ssems[r], add=True)

            def inner(t, _):
                proc(4 * t, 0)
                proc(4 * t + 1, 1)
                proc(4 * t + 2, 2)
                proc(4 * t + 3, 3)
                return 0

            lax.fori_loop(0, g // 4, inner, 0)

        ls = (nseg - 1) % 2
        pltpu.make_async_copy(rows.at[1], acc.at[seg.at[ls, g - 3, 1]],
                              ssems[1]).wait()
        pltpu.make_async_copy(rows.at[2], acc.at[seg.at[ls, g - 2, 1]],
                              ssems[2]).wait()
        pltpu.make_async_copy(rows.at[3], acc.at[seg.at[ls, g - 1, 1]],
                              ssems[3]).wait()
        plsc.subcore_barrier()
        pltpu.sync_copy(acc.at[pl.ds(base, _RPT)],
                        out_hbm.at[cid, pl.ds(base, _RPT)])

    return _conv


_C64 = 125
_K64 = _E // (_NW * _C64)
_G64 = 16

_conv128 = _make_conv(_D_HID, _C, _K, _G)
_conv64 = _make_conv(_D_OUT, _C64, _K64, _G64)


_R = 1000


def _pre_body(x_ref, wg1_ref, d0_ref, d1_ref, u1_ref, dinv_ref):
    x = x_ref[...]
    deg = d0_ref[0, 0, :] + d1_ref[0, 0, :] + 1.0
    dinv = lax.rsqrt(deg)
    dinv_ref[0, 0, :] = dinv
    u1_ref[...] = jnp.dot(x, wg1_ref[...],
                          preferred_element_type=jnp.float32) * dinv[:, None]


_pre_call = pl.pallas_call(
    _pre_body,
    grid=(_N // _R,),
    in_specs=[
        pl.BlockSpec((_R, _D_IN), lambda i: (i, 0)),
        pl.BlockSpec((_D_IN, _D_HID), lambda i: (0, 0)),
        pl.BlockSpec((1, 1, _R), lambda i: (i, 0, 0)),
        pl.BlockSpec((1, 1, _R), lambda i: (i, 0, 0)),
    ],
    out_specs=[
        pl.BlockSpec((_R, _D_HID), lambda i: (i, 0)),
        pl.BlockSpec((1, 1, _R), lambda i: (i, 0, 0)),
    ],
    out_shape=[
        jax.ShapeDtypeStruct((_N, _D_HID), jnp.float32),
        jax.ShapeDtypeStruct((_N // _R, 1, _R), jnp.float32),
    ],
)


def _mid_body(s1a_ref, s1b_ref, u1_ref, dinv_ref, bg1_ref, wg2_ref, u2_ref):
    dinv = dinv_ref[0, 0, :]
    z = (s1a_ref[...] + s1b_ref[...] + u1_ref[...]) * dinv[:, None] + bg1_ref[...][None, :]
    z = jnp.maximum(z, 0.0)
    u2_ref[...] = jnp.dot(z, wg2_ref[...],
                          preferred_element_type=jnp.float32) * dinv[:, None]


_mid_call = pl.pallas_call(
    _mid_body,
    grid=(_N // _R,),
    in_specs=[
        pl.BlockSpec((_R, _D_HID), lambda i: (i, 0)),
        pl.BlockSpec((_R, _D_HID), lambda i: (i, 0)),
        pl.BlockSpec((_R, _D_HID), lambda i: (i, 0)),
        pl.BlockSpec((1, 1, _R), lambda i: (i, 0, 0)),
        pl.BlockSpec((_D_HID,), lambda i: (0,)),
        pl.BlockSpec((_D_HID, _D_OUT), lambda i: (0, 0)),
    ],
    out_specs=pl.BlockSpec((_R, _D_OUT), lambda i: (i, 0)),
    out_shape=jax.ShapeDtypeStruct((_N, _D_OUT), jnp.float32),
)


def _post_body(s2a_ref, s2b_ref, u2_ref, dinv_ref, bg2_ref, x_ref, wm1_ref,
               bm1_ref, wm2_ref, bm2_ref, spi_ref, logt_ref, out_ref):
    dinv = dinv_ref[0, 0, :]
    z_gnn = ((s2a_ref[...] + s2b_ref[...] + u2_ref[...]) * dinv[:, None]
             + bg2_ref[...][None, :])
    m = jnp.maximum(jnp.dot(x_ref[...], wm1_ref[...],
                            preferred_element_type=jnp.float32)
                    + bm1_ref[...][None, :], 0.0)
    z_mlp = jnp.dot(m, wm2_ref[...],
                    preferred_element_type=jnp.float32) + bm2_ref[...][None, :]
    beta = jax.nn.sigmoid((spi_ref[0, 0] - _TAU) * jnp.exp(-logt_ref[0, 0]))
    out_ref[...] = beta * z_gnn + (1.0 - beta) * z_mlp


_post_call = pl.pallas_call(
    _post_body,
    grid=(_N // _R,),
    in_specs=[
        pl.BlockSpec((_R, _D_OUT), lambda i: (i, 0)),
        pl.BlockSpec((_R, _D_OUT), lambda i: (i, 0)),
        pl.BlockSpec((_R, _D_OUT), lambda i: (i, 0)),
        pl.BlockSpec((1, 1, _R), lambda i: (i, 0, 0)),
        pl.BlockSpec((_D_OUT,), lambda i: (0,)),
        pl.BlockSpec((_R, _D_IN), lambda i: (i, 0)),
        pl.BlockSpec((_D_IN, _D_HID), lambda i: (0, 0)),
        pl.BlockSpec((_D_HID,), lambda i: (0,)),
        pl.BlockSpec((_D_HID, _D_OUT), lambda i: (0, 0)),
        pl.BlockSpec((_D_OUT,), lambda i: (0,)),
        pl.BlockSpec((1, 1), lambda i: (0, 0)),
        pl.BlockSpec((1, 1), lambda i: (0, 0)),
    ],
    out_specs=pl.BlockSpec((_R, _D_OUT), lambda i: (i, 0)),
    out_shape=jax.ShapeDtypeStruct((_N, _D_OUT), jnp.float32),
)


def kernel(x, edge_index, spi, W_g1, b_g1, W_g2, b_g2, W_m1, b_m1, W_m2, b_m2, log_T):
    srcr = edge_index[0].reshape(_NW, _K, _C)
    dstr = edge_index[1].reshape(_NW, _K, _C)
    edges = jnp.stack([srcr, dstr], axis=2)
    src64 = edge_index[0].reshape(_NW, _K64, _C64)
    dst64 = edge_index[1].reshape(_NW, _K64, _C64)
    edges64 = jnp.stack([src64, dst64], axis=2)
    dstd = edge_index[1].reshape(_NW, _KD, _CD)
    degp = _deg_kernel(dstd)
    d0 = degp[:_N].reshape(_N // _R, 1, _R)
    d1 = degp[_NPAD:_NPAD + _N].reshape(_N // _R, 1, _R)
    u1, dinv = _pre_call(x, W_g1, d0, d1)
    s1 = _conv128(u1, edges)
    u2 = _mid_call(s1[0, :_N], s1[1, :_N], u1, dinv, b_g1, W_g2)
    s2 = _conv64(u2, edges64)
    out = _post_call(s2[0, :_N], s2[1, :_N], u2, dinv, b_g2, x, W_m1, b_m1,
                     W_m2, b_m2, spi.reshape(1, 1), log_T.reshape(1, 1))
    return out
